# Initial kernel scaffold; baseline (speedup 1.0000x reference)
import jax
import jax.numpy as jnp
from jax.experimental import pallas as pl


def kernel(x, edge_index, W1, b1, W2, b2, W3, b3):
    raise NotImplementedError("write your pallas kernel here")


# speedup vs baseline: 25.5932x; 25.5932x over previous
import functools

import jax
import jax.numpy as jnp
from jax import lax
from jax.experimental import pallas as pl
from jax.experimental.pallas import tpu as pltpu
from jax.experimental.pallas import tpu_sc as plsc

N = 10000
NC, NS = 2, 16
NW = NC * NS
CHUNK = 128
RPT = 632
N_ACC = NS * RPT
DEG_ROWS = 640
DEG_RPT = DEG_ROWS // NS
DEG_CH = DEG_ROWS // CHUNK

RB = 1000
GRID = N // RB


def _sc_mesh():
    return plsc.VectorSubcoreMesh(core_axis_name="c", subcore_axis_name="s")


DEGW = DEG_ROWS * 16


def _make_deg(ept):
    steps = ept // 16

    @functools.partial(
        pl.kernel,
        out_type=jax.ShapeDtypeStruct((NW, DEGW), jnp.float32),
        mesh=_sc_mesh(),
        compiler_params=pltpu.CompilerParams(needs_layout_passes=False),
        scratch_types=[
            pltpu.VMEM((ept,), jnp.int32),
            pltpu.VMEM((DEGW,), jnp.float32),
        ],
    )
    def deg_kernel(dst_hbm, zeros_hbm, out_hbm, idxv, degv):
        c = lax.axis_index("c")
        s = lax.axis_index("s")
        wid = c * NS + s
        pltpu.sync_copy(zeros_hbm, degv)
        pltpu.sync_copy(dst_hbm.at[wid], idxv)
        ones = jnp.full((16,), 1.0, jnp.float32)

        def step(i, carry):
            idx = idxv[pl.ds(i * 16, 16)]
            plsc.addupdate_scatter(degv, [idx], ones)
            return carry

        lax.fori_loop(0, steps, step, 0)
        pltpu.sync_copy(degv, out_hbm.at[wid])

    return deg_kernel


def _make_scatter(f, nch):
    @functools.partial(
        pl.kernel,
        out_type=jax.ShapeDtypeStruct((NC, N_ACC, f), jnp.float32),
        mesh=_sc_mesh(),
        compiler_params=pltpu.CompilerParams(use_tc_tiling_on_sc=False),
        scratch_types=[
            pltpu.VMEM((nch, CHUNK), jnp.int32),
            pltpu.VMEM((nch, CHUNK), jnp.int32),
            pltpu.VMEM((CHUNK, f), jnp.float32),
            pltpu.VMEM_SHARED((N_ACC, f), jnp.float32),
            pltpu.SemaphoreType.DMA,
        ],
    )
    def scatter_kernel(z_hbm, src_hbm, dst_hbm, zeros_hbm, out_hbm,
                       idxs, idxd, rows, acc, sem):
        c = lax.axis_index("c")
        s = lax.axis_index("s")
        wid = c * NS + s
        pltpu.sync_copy(zeros_hbm.at[pl.ds(s * RPT, RPT)],
                        acc.at[pl.ds(s * RPT, RPT)])
        pltpu.sync_copy(src_hbm.at[wid], idxs)
        pltpu.sync_copy(dst_hbm.at[wid], idxd)
        plsc.subcore_barrier()

        def chunk(j, carry):
            pltpu.async_copy(z_hbm.at[idxs.at[j]], rows, sem).wait()
            pltpu.sync_copy(rows, acc.at[idxd.at[j]], add=True)
            return carry

        lax.fori_loop(0, nch, chunk, 0)
        plsc.subcore_barrier()
        pltpu.sync_copy(acc.at[pl.ds(s * RPT, RPT)],
                        out_hbm.at[c, pl.ds(s * RPT, RPT)])

    return scatter_kernel


def _row(f):
    return pl.BlockSpec((RB, f), lambda i: (i, 0))


def _full(r, c):
    return pl.BlockSpec((r, c), lambda i: (0, 0))


def _prep_body(dp, x, w1, dinv_o, z1_o):
    deg = jnp.sum(dp[...], axis=1, keepdims=True) + 1.0
    dinv = lax.rsqrt(deg)
    dinv_o[...] = dinv
    z1_o[...] = dinv * jnp.dot(x[...], w1[...],
                               preferred_element_type=jnp.float32)


_prep = pl.pallas_call(
    _prep_body,
    grid=(GRID,),
    in_specs=[_row(NW), _row(200), _full(200, 32)],
    out_specs=[_row(1), _row(32)],
    out_shape=[jax.ShapeDtypeStruct((N, 1), jnp.float32),
               jax.ShapeDtypeStruct((N, 32), jnp.float32)],
)


def _mid1_body(a0, a1, z1, dinv, b1, w2, z2_o):
    h = jnp.maximum(dinv[...] * (a0[...] + a1[...] + z1[...]) + b1[...], 0.0)
    z2_o[...] = dinv[...] * jnp.dot(h, w2[...],
                                    preferred_element_type=jnp.float32)


_mid1 = pl.pallas_call(
    _mid1_body,
    grid=(GRID,),
    in_specs=[_row(32), _row(32), _row(32), _row(1), _full(1, 32), _full(32, 16)],
    out_specs=_row(16),
    out_shape=jax.ShapeDtypeStruct((N, 16), jnp.float32),
)


def _mid2_body(a0, a1, z2, dinv, b2, z3_o):
    h = jnp.maximum(dinv[...] * (a0[...] + a1[...] + z2[...]) + b2[...], 0.0)
    z3_o[...] = dinv[...] * h


_mid2 = pl.pallas_call(
    _mid2_body,
    grid=(GRID,),
    in_specs=[_row(16), _row(16), _row(16), _row(1), _full(1, 16)],
    out_specs=_row(16),
    out_shape=jax.ShapeDtypeStruct((N, 16), jnp.float32),
)


def _final_body(a0, a1, z3, dinv, w3, b3, out_o):
    p = dinv[...] * (a0[...] + a1[...] + z3[...])
    h = jnp.dot(p, w3[...], preferred_element_type=jnp.float32) + b3[...]
    m = jnp.max(h, axis=1, keepdims=True)
    e = h - m
    out_o[...] = e - jnp.log(jnp.sum(jnp.exp(e), axis=1, keepdims=True))


_final = pl.pallas_call(
    _final_body,
    grid=(GRID,),
    in_specs=[_row(16), _row(16), _row(16), _row(1), _full(16, 200), _full(1, 200)],
    out_specs=_row(200),
    out_shape=jax.ShapeDtypeStruct((N, 200), jnp.float32),
)


def kernel(x, edge_index, W1, b1, W2, b2, W3, b3):
    e = edge_index.shape[1]
    ept = -(-e // (NW * CHUNK)) * CHUNK
    nch = ept // CHUNK
    e_pad = ept * NW

    ei = edge_index.astype(jnp.int32)
    src = jnp.concatenate([ei[0], jnp.zeros((e_pad - e,), jnp.int32)])
    dst = jnp.concatenate([ei[1], jnp.full((e_pad - e,), N, jnp.int32)])
    src_r = src.reshape(NW, nch, CHUNK)
    dst_r = dst.reshape(NW, nch, CHUNK)
    dst_f = dst.reshape(NW, ept)
    zdeg = jnp.zeros((DEGW,), jnp.float32)
    z16 = jnp.zeros((N_ACC, 16), jnp.float32)
    z32 = jnp.zeros((N_ACC, 32), jnp.float32)

    degp = _make_deg(ept)(dst_f, zdeg)
    dp = degp[:, :N].T

    dinv, zr1 = _prep(dp, x, W1)

    a = _make_scatter(32, nch)(zr1, src_r, dst_r, z32)
    zr2 = _mid1(a[0, :N], a[1, :N], zr1, dinv, b1.reshape(1, 32), W2)

    a = _make_scatter(16, nch)(zr2, src_r, dst_r, z16)
    zr3 = _mid2(a[0, :N], a[1, :N], zr2, dinv, b2.reshape(1, 16))

    a = _make_scatter(16, nch)(zr3, src_r, dst_r, z16)
    return _final(a[0, :N], a[1, :N], zr3, dinv, W3, b3.reshape(1, 200))


# speedup vs baseline: 28.9909x; 1.1328x over previous
import functools

import jax
import jax.numpy as jnp
from jax import lax
from jax.experimental import pallas as pl
from jax.experimental.pallas import tpu as pltpu
from jax.experimental.pallas import tpu_sc as plsc

N = 10000
NC, NS = 2, 16
NW = NC * NS
CHUNK = 128
RPT = 632
N_ACC = NS * RPT
DEG_ROWS = 640
DEG_RPT = DEG_ROWS // NS
DEG_CH = DEG_ROWS // CHUNK

RB = 1000
GRID = N // RB


def _sc_mesh():
    return plsc.VectorSubcoreMesh(core_axis_name="c", subcore_axis_name="s")


DEGW = DEG_ROWS * 16


def _make_deg(ept):
    steps = ept // 16

    @functools.partial(
        pl.kernel,
        out_type=jax.ShapeDtypeStruct((NW, DEGW), jnp.float32),
        mesh=_sc_mesh(),
        compiler_params=pltpu.CompilerParams(needs_layout_passes=False),
        scratch_types=[
            pltpu.VMEM((ept,), jnp.int32),
            pltpu.VMEM((DEGW,), jnp.float32),
        ],
    )
    def deg_kernel(dst_hbm, zeros_hbm, out_hbm, idxv, degv):
        c = lax.axis_index("c")
        s = lax.axis_index("s")
        wid = c * NS + s
        pltpu.sync_copy(zeros_hbm, degv)
        pltpu.sync_copy(dst_hbm.at[wid], idxv)
        ones = jnp.full((16,), 1.0, jnp.float32)

        def step(i, carry):
            idx = idxv[pl.ds(i * 16, 16)]
            plsc.addupdate_scatter(degv, [idx], ones)
            return carry

        lax.fori_loop(0, steps, step, 0)
        pltpu.sync_copy(degv, out_hbm.at[wid])

    return deg_kernel


G = 4


def _make_scatter(f, nch):
    ngrp = nch // (2 * G)

    @functools.partial(
        pl.kernel,
        out_type=jax.ShapeDtypeStruct((NC, N_ACC, f), jnp.float32),
        mesh=_sc_mesh(),
        compiler_params=pltpu.CompilerParams(use_tc_tiling_on_sc=False),
        scratch_types=[
            pltpu.VMEM((nch, CHUNK), jnp.int32),
            pltpu.VMEM((nch, CHUNK), jnp.int32),
            pltpu.VMEM((G * CHUNK, f), jnp.float32),
            pltpu.VMEM((G * CHUNK, f), jnp.float32),
            pltpu.VMEM_SHARED((N_ACC, f), jnp.float32),
            pltpu.SemaphoreType.DMA,
            pltpu.SemaphoreType.DMA,
            pltpu.SemaphoreType.DMA,
            pltpu.SemaphoreType.DMA,
        ],
    )
    def scatter_kernel(z_hbm, src_hbm, dst_hbm, zeros_hbm, out_hbm,
                       idxs, idxd, rows_a, rows_b, acc,
                       sem_ga, sem_gb, sem_sa, sem_sb):
        c = lax.axis_index("c")
        s = lax.axis_index("s")
        wid = c * NS + s
        pltpu.sync_copy(zeros_hbm.at[pl.ds(s * RPT, RPT)],
                        acc.at[pl.ds(s * RPT, RPT)])
        pltpu.sync_copy(src_hbm.at[wid], idxs)
        pltpu.sync_copy(dst_hbm.at[wid], idxd)
        plsc.subcore_barrier()

        def gath(j, t, buf, sem):
            return pltpu.make_async_copy(
                z_hbm.at[idxs.at[j + t]],
                buf.at[pl.ds(t * CHUNK, CHUNK)], sem)

        def scat_start(j, t, buf, sem):
            pltpu.async_copy(buf.at[pl.ds(t * CHUNK, CHUNK)],
                             acc.at[idxd.at[j + t]], sem, add=True)

        def scat_wait(j, t, buf, sem):
            pltpu.make_async_copy(buf.at[pl.ds(t * CHUNK, CHUNK)],
                                  acc.at[idxd.at[j + t]], sem).wait()

        for t in range(G):
            gath(0, t, rows_a, sem_ga).start()

        def grp(k, carry):
            ja = k * 2 * G
            jb = ja + G
            for t in range(G):
                gath(ja, t, rows_a, sem_ga).wait()

            @pl.when(k > 0)
            def _():
                for t in range(G):
                    scat_wait(ja - G, t, rows_b, sem_sb)

            for t in range(G):
                gath(jb, t, rows_b, sem_gb).start()
            for t in range(G):
                scat_start(ja, t, rows_a, sem_sa)
            for t in range(G):
                gath(jb, t, rows_b, sem_gb).wait()
            for t in range(G):
                scat_wait(ja, t, rows_a, sem_sa)

            @pl.when(k + 1 < ngrp)
            def _():
                for t in range(G):
                    gath(ja + 2 * G, t, rows_a, sem_ga).start()

            for t in range(G):
                scat_start(jb, t, rows_b, sem_sb)
            return carry

        lax.fori_loop(0, ngrp, grp, 0)
        for t in range(G):
            scat_wait((ngrp - 1) * 2 * G + G, t, rows_b, sem_sb)
        plsc.subcore_barrier()
        pltpu.sync_copy(acc.at[pl.ds(s * RPT, RPT)],
                        out_hbm.at[c, pl.ds(s * RPT, RPT)])

    return scatter_kernel


def _row(f):
    return pl.BlockSpec((RB, f), lambda i: (i, 0))


def _full(r, c):
    return pl.BlockSpec((r, c), lambda i: (0, 0))


def _prep_body(dp, x, w1, dinv_o, z1_o):
    deg = jnp.sum(dp[...], axis=1, keepdims=True) + 1.0
    dinv = lax.rsqrt(deg)
    dinv_o[...] = dinv
    z1_o[...] = dinv * jnp.dot(x[...], w1[...],
                               preferred_element_type=jnp.float32)


_prep = pl.pallas_call(
    _prep_body,
    grid=(GRID,),
    in_specs=[_row(NW), _row(200), _full(200, 32)],
    out_specs=[_row(1), _row(32)],
    out_shape=[jax.ShapeDtypeStruct((N, 1), jnp.float32),
               jax.ShapeDtypeStruct((N, 32), jnp.float32)],
)


def _mid1_body(a0, a1, z1, dinv, b1, w2, z2_o):
    h = jnp.maximum(dinv[...] * (a0[...] + a1[...] + z1[...]) + b1[...], 0.0)
    z2_o[...] = dinv[...] * jnp.dot(h, w2[...],
                                    preferred_element_type=jnp.float32)


_mid1 = pl.pallas_call(
    _mid1_body,
    grid=(GRID,),
    in_specs=[_row(32), _row(32), _row(32), _row(1), _full(1, 32), _full(32, 16)],
    out_specs=_row(16),
    out_shape=jax.ShapeDtypeStruct((N, 16), jnp.float32),
)


def _mid2_body(a0, a1, z2, dinv, b2, z3_o):
    h = jnp.maximum(dinv[...] * (a0[...] + a1[...] + z2[...]) + b2[...], 0.0)
    z3_o[...] = dinv[...] * h


_mid2 = pl.pallas_call(
    _mid2_body,
    grid=(GRID,),
    in_specs=[_row(16), _row(16), _row(16), _row(1), _full(1, 16)],
    out_specs=_row(16),
    out_shape=jax.ShapeDtypeStruct((N, 16), jnp.float32),
)


def _final_body(a0, a1, z3, dinv, w3, b3, out_o):
    p = dinv[...] * (a0[...] + a1[...] + z3[...])
    h = jnp.dot(p, w3[...], preferred_element_type=jnp.float32) + b3[...]
    m = jnp.max(h, axis=1, keepdims=True)
    e = h - m
    out_o[...] = e - jnp.log(jnp.sum(jnp.exp(e), axis=1, keepdims=True))


_final = pl.pallas_call(
    _final_body,
    grid=(GRID,),
    in_specs=[_row(16), _row(16), _row(16), _row(1), _full(16, 200), _full(1, 200)],
    out_specs=_row(200),
    out_shape=jax.ShapeDtypeStruct((N, 200), jnp.float32),
)


def kernel(x, edge_index, W1, b1, W2, b2, W3, b3):
    e = edge_index.shape[1]
    nch = -(-(-(-e // (NW * CHUNK))) // (2 * G)) * (2 * G)
    ept = nch * CHUNK
    e_pad = ept * NW

    ei = edge_index.astype(jnp.int32)
    src = jnp.concatenate([ei[0], jnp.zeros((e_pad - e,), jnp.int32)])
    dst = jnp.concatenate([ei[1], jnp.full((e_pad - e,), N, jnp.int32)])
    src_r = src.reshape(NW, nch, CHUNK)
    dst_r = dst.reshape(NW, nch, CHUNK)
    dst_f = dst.reshape(NW, ept)
    zdeg = jnp.zeros((DEGW,), jnp.float32)
    z16 = jnp.zeros((N_ACC, 16), jnp.float32)
    z32 = jnp.zeros((N_ACC, 32), jnp.float32)

    degp = _make_deg(ept)(dst_f, zdeg)
    dp = degp[:, :N].T

    dinv, zr1 = _prep(dp, x, W1)

    a = _make_scatter(32, nch)(zr1, src_r, dst_r, z32)
    zr2 = _mid1(a[0, :N], a[1, :N], zr1, dinv, b1.reshape(1, 32), W2)

    a = _make_scatter(16, nch)(zr2, src_r, dst_r, z16)
    zr3 = _mid2(a[0, :N], a[1, :N], zr2, dinv, b2.reshape(1, 16))

    a = _make_scatter(16, nch)(zr3, src_r, dst_r, z16)
    return _final(a[0, :N], a[1, :N], zr3, dinv, W3, b3.reshape(1, 200))


# speedup vs baseline: 29.6769x; 1.0237x over previous
import functools

import jax
import jax.numpy as jnp
from jax import lax
from jax.experimental import pallas as pl
from jax.experimental.pallas import tpu as pltpu
from jax.experimental.pallas import tpu_sc as plsc

N = 10000
NC, NS = 2, 16
NW = NC * NS
CHUNK = 128
RPT = 632
N_ACC = NS * RPT
DEG_ROWS = 640
DEG_RPT = DEG_ROWS // NS
DEG_CH = DEG_ROWS // CHUNK

RB = 1000
GRID = N // RB


def _sc_mesh():
    return plsc.VectorSubcoreMesh(core_axis_name="c", subcore_axis_name="s")


DEGW = DEG_ROWS * 16


def _make_deg(ept):
    steps = ept // 16

    @functools.partial(
        pl.kernel,
        out_type=jax.ShapeDtypeStruct((NW, DEGW), jnp.float32),
        mesh=_sc_mesh(),
        compiler_params=pltpu.CompilerParams(needs_layout_passes=False),
        scratch_types=[
            pltpu.VMEM((ept,), jnp.int32),
            pltpu.VMEM((DEGW,), jnp.float32),
        ],
    )
    def deg_kernel(dst_hbm, zeros_hbm, out_hbm, idxv, degv):
        c = lax.axis_index("c")
        s = lax.axis_index("s")
        wid = c * NS + s
        pltpu.sync_copy(zeros_hbm, degv)
        pltpu.sync_copy(dst_hbm.at[wid], idxv)
        ones = jnp.full((16,), 1.0, jnp.float32)

        def step(i, carry):
            idx = idxv[pl.ds(i * 16, 16)]
            plsc.addupdate_scatter(degv, [idx], ones)
            return carry

        lax.fori_loop(0, steps, step, 0)
        pltpu.sync_copy(degv, out_hbm.at[wid])

    return deg_kernel


G = 8
WIDE = G * CHUNK


def _make_scatter(f, nch):
    ngrp = nch // (2 * G)

    @functools.partial(
        pl.kernel,
        out_type=jax.ShapeDtypeStruct((NC, N_ACC, f), jnp.float32),
        mesh=_sc_mesh(),
        compiler_params=pltpu.CompilerParams(use_tc_tiling_on_sc=False),
        scratch_types=[
            pltpu.VMEM((nch // G, WIDE), jnp.int32),
            pltpu.VMEM((nch, CHUNK), jnp.int32),
            pltpu.VMEM((WIDE, f), jnp.float32),
            pltpu.VMEM((WIDE, f), jnp.float32),
            pltpu.VMEM_SHARED((N_ACC, f), jnp.float32),
            pltpu.SemaphoreType.DMA,
            pltpu.SemaphoreType.DMA,
            pltpu.SemaphoreType.DMA,
            pltpu.SemaphoreType.DMA,
        ],
    )
    def scatter_kernel(z_hbm, src_hbm, dst_hbm, zeros_hbm, out_hbm,
                       idxs, idxd, rows_a, rows_b, acc,
                       sem_ga, sem_gb, sem_sa, sem_sb):
        c = lax.axis_index("c")
        s = lax.axis_index("s")
        wid = c * NS + s
        pltpu.sync_copy(zeros_hbm.at[pl.ds(s * RPT, RPT)],
                        acc.at[pl.ds(s * RPT, RPT)])
        pltpu.sync_copy(src_hbm.at[wid], idxs)
        pltpu.sync_copy(dst_hbm.at[wid], idxd)
        plsc.subcore_barrier()

        def gath(g, buf, sem):
            return pltpu.make_async_copy(z_hbm.at[idxs.at[g]], buf, sem)

        def scat_start(j, t, buf, sem):
            pltpu.async_copy(buf.at[pl.ds(t * CHUNK, CHUNK)],
                             acc.at[idxd.at[j + t]], sem, add=True)

        def scat_wait(j, t, buf, sem):
            pltpu.make_async_copy(buf.at[pl.ds(t * CHUNK, CHUNK)],
                                  acc.at[idxd.at[j + t]], sem).wait()

        gath(0, rows_a, sem_ga).start()

        def grp(k, carry):
            ga = 2 * k
            ja = ga * G
            jb = ja + G
            gath(ga, rows_a, sem_ga).wait()

            @pl.when(k > 0)
            def _():
                for t in range(G):
                    scat_wait(ja - G, t, rows_b, sem_sb)

            gath(ga + 1, rows_b, sem_gb).start()
            for t in range(G):
                scat_start(ja, t, rows_a, sem_sa)
            gath(ga + 1, rows_b, sem_gb).wait()
            for t in range(G):
                scat_wait(ja, t, rows_a, sem_sa)

            @pl.when(k + 1 < ngrp)
            def _():
                gath(ga + 2, rows_a, sem_ga).start()

            for t in range(G):
                scat_start(jb, t, rows_b, sem_sb)
            return carry

        lax.fori_loop(0, ngrp, grp, 0)
        for t in range(G):
            scat_wait((ngrp - 1) * 2 * G + G, t, rows_b, sem_sb)
        plsc.subcore_barrier()
        pltpu.sync_copy(acc.at[pl.ds(s * RPT, RPT)],
                        out_hbm.at[c, pl.ds(s * RPT, RPT)])

    return scatter_kernel


def _row(f):
    return pl.BlockSpec((RB, f), lambda i: (i, 0))


def _full(r, c):
    return pl.BlockSpec((r, c), lambda i: (0, 0))


def _prep_body(dp, x, w1, dinv_o, z1_o):
    deg = jnp.sum(dp[...], axis=1, keepdims=True) + 1.0
    dinv = lax.rsqrt(deg)
    dinv_o[...] = dinv
    z1_o[...] = dinv * jnp.dot(x[...], w1[...],
                               preferred_element_type=jnp.float32)


_prep = pl.pallas_call(
    _prep_body,
    grid=(GRID,),
    in_specs=[_row(NW), _row(200), _full(200, 32)],
    out_specs=[_row(1), _row(32)],
    out_shape=[jax.ShapeDtypeStruct((N, 1), jnp.float32),
               jax.ShapeDtypeStruct((N, 32), jnp.float32)],
)


def _mid1_body(a0, a1, z1, dinv, b1, w2, z2_o):
    h = jnp.maximum(dinv[...] * (a0[...] + a1[...] + z1[...]) + b1[...], 0.0)
    z2_o[...] = dinv[...] * jnp.dot(h, w2[...],
                                    preferred_element_type=jnp.float32)


_mid1 = pl.pallas_call(
    _mid1_body,
    grid=(GRID,),
    in_specs=[_row(32), _row(32), _row(32), _row(1), _full(1, 32), _full(32, 16)],
    out_specs=_row(16),
    out_shape=jax.ShapeDtypeStruct((N, 16), jnp.float32),
)


def _mid2_body(a0, a1, z2, dinv, b2, z3_o):
    h = jnp.maximum(dinv[...] * (a0[...] + a1[...] + z2[...]) + b2[...], 0.0)
    z3_o[...] = dinv[...] * h


_mid2 = pl.pallas_call(
    _mid2_body,
    grid=(GRID,),
    in_specs=[_row(16), _row(16), _row(16), _row(1), _full(1, 16)],
    out_specs=_row(16),
    out_shape=jax.ShapeDtypeStruct((N, 16), jnp.float32),
)


def _final_body(a0, a1, z3, dinv, w3, b3, out_o):
    p = dinv[...] * (a0[...] + a1[...] + z3[...])
    h = jnp.dot(p, w3[...], preferred_element_type=jnp.float32) + b3[...]
    m = jnp.max(h, axis=1, keepdims=True)
    e = h - m
    out_o[...] = e - jnp.log(jnp.sum(jnp.exp(e), axis=1, keepdims=True))


_final = pl.pallas_call(
    _final_body,
    grid=(GRID,),
    in_specs=[_row(16), _row(16), _row(16), _row(1), _full(16, 200), _full(1, 200)],
    out_specs=_row(200),
    out_shape=jax.ShapeDtypeStruct((N, 200), jnp.float32),
)


def kernel(x, edge_index, W1, b1, W2, b2, W3, b3):
    e = edge_index.shape[1]
    nch = -(-(-(-e // (NW * CHUNK))) // (2 * G)) * (2 * G)
    ept = nch * CHUNK
    e_pad = ept * NW

    ei = edge_index.astype(jnp.int32)
    src = jnp.concatenate([ei[0], jnp.zeros((e_pad - e,), jnp.int32)])
    dst = jnp.concatenate([ei[1], jnp.full((e_pad - e,), N, jnp.int32)])
    src_r = src.reshape(NW, nch // G, WIDE)
    dst_r = dst.reshape(NW, nch, CHUNK)
    dst_f = dst.reshape(NW, ept)
    zdeg = jnp.zeros((DEGW,), jnp.float32)
    z16 = jnp.zeros((N_ACC, 16), jnp.float32)
    z32 = jnp.zeros((N_ACC, 32), jnp.float32)

    degp = _make_deg(ept)(dst_f, zdeg)
    dp = degp[:, :N].T

    dinv, zr1 = _prep(dp, x, W1)

    a = _make_scatter(32, nch)(zr1, src_r, dst_r, z32)
    zr2 = _mid1(a[0, :N], a[1, :N], zr1, dinv, b1.reshape(1, 32), W2)

    a = _make_scatter(16, nch)(zr2, src_r, dst_r, z16)
    zr3 = _mid2(a[0, :N], a[1, :N], zr2, dinv, b2.reshape(1, 16))

    a = _make_scatter(16, nch)(zr3, src_r, dst_r, z16)
    return _final(a[0, :N], a[1, :N], zr3, dinv, W3, b3.reshape(1, 200))


# speedup vs baseline: 43.6399x; 1.4705x over previous
import functools

import jax
import jax.numpy as jnp
from jax import lax
from jax.experimental import pallas as pl
from jax.experimental.pallas import tpu as pltpu
from jax.experimental.pallas import tpu_sc as plsc

N = 10000
NC, NS = 2, 16
NW = NC * NS
CHUNK = 128
RPT = 632
N_ACC = NS * RPT
DEG_ROWS = 640
DEG_RPT = DEG_ROWS // NS
DEG_CH = DEG_ROWS // CHUNK

RB = 1000
GRID = N // RB


def _sc_mesh():
    return plsc.VectorSubcoreMesh(core_axis_name="c", subcore_axis_name="s")


DEGW = DEG_ROWS * 16


def _make_deg(ept):
    steps = ept // 16

    @functools.partial(
        pl.kernel,
        out_type=jax.ShapeDtypeStruct((NW, DEGW), jnp.float32),
        mesh=_sc_mesh(),
        compiler_params=pltpu.CompilerParams(needs_layout_passes=False),
        scratch_types=[
            pltpu.VMEM((ept,), jnp.int32),
            pltpu.VMEM((DEGW,), jnp.float32),
        ],
    )
    def deg_kernel(dst_hbm, zeros_hbm, out_hbm, idxv, degv):
        c = lax.axis_index("c")
        s = lax.axis_index("s")
        wid = c * NS + s
        pltpu.sync_copy(zeros_hbm, degv)
        pltpu.sync_copy(dst_hbm.at[wid], idxv)
        ones = jnp.full((16,), 1.0, jnp.float32)

        def step(i, carry):
            idx = idxv[pl.ds(i * 16, 16)]
            plsc.addupdate_scatter(degv, [idx], ones)
            return carry

        lax.fori_loop(0, steps, step, 0)
        pltpu.sync_copy(degv, out_hbm.at[wid])

    return deg_kernel


G = 8
WIDE = G * CHUNK


def _make_scatter(f, nch):
    ngrp = nch // (2 * G)

    @functools.partial(
        pl.kernel,
        out_type=jax.ShapeDtypeStruct((NC, N_ACC, f), jnp.float32),
        mesh=_sc_mesh(),
        compiler_params=pltpu.CompilerParams(use_tc_tiling_on_sc=False),
        scratch_types=[
            pltpu.VMEM((nch // G, WIDE), jnp.int32),
            pltpu.VMEM((nch, CHUNK), jnp.int32),
            pltpu.VMEM((WIDE, f), jnp.float32),
            pltpu.VMEM((WIDE, f), jnp.float32),
            pltpu.VMEM_SHARED((N_ACC, f), jnp.float32),
            pltpu.VMEM_SHARED((N_ACC, f), jnp.float32),
            pltpu.SemaphoreType.DMA,
            pltpu.SemaphoreType.DMA,
            pltpu.SemaphoreType.DMA,
            pltpu.SemaphoreType.DMA,
        ],
    )
    def scatter_kernel(z_hbm, src_hbm, dst_hbm, zeros_hbm, out_hbm,
                       idxs, idxd, rows_a, rows_b, acc, zs,
                       sem_ga, sem_gb, sem_sa, sem_sb):
        c = lax.axis_index("c")
        s = lax.axis_index("s")
        wid = c * NS + s
        pltpu.sync_copy(zeros_hbm.at[pl.ds(s * RPT, RPT)],
                        acc.at[pl.ds(s * RPT, RPT)])
        pltpu.sync_copy(z_hbm.at[pl.ds(s * RPT, RPT)],
                        zs.at[pl.ds(s * RPT, RPT)])
        pltpu.sync_copy(src_hbm.at[wid], idxs)
        pltpu.sync_copy(dst_hbm.at[wid], idxd)
        plsc.subcore_barrier()

        def gath(g, buf, sem):
            return pltpu.make_async_copy(zs.at[idxs.at[g]], buf, sem)

        def scat_start(j, t, buf, sem):
            pltpu.async_copy(buf.at[pl.ds(t * CHUNK, CHUNK)],
                             acc.at[idxd.at[j + t]], sem, add=True)

        def scat_wait(j, t, buf, sem):
            pltpu.make_async_copy(buf.at[pl.ds(t * CHUNK, CHUNK)],
                                  acc.at[idxd.at[j + t]], sem).wait()

        gath(0, rows_a, sem_ga).start()

        def grp(k, carry):
            ga = 2 * k
            ja = ga * G
            jb = ja + G
            gath(ga, rows_a, sem_ga).wait()

            @pl.when(k > 0)
            def _():
                for t in range(G):
                    scat_wait(ja - G, t, rows_b, sem_sb)

            gath(ga + 1, rows_b, sem_gb).start()
            for t in range(G):
                scat_start(ja, t, rows_a, sem_sa)
            gath(ga + 1, rows_b, sem_gb).wait()
            for t in range(G):
                scat_wait(ja, t, rows_a, sem_sa)

            @pl.when(k + 1 < ngrp)
            def _():
                gath(ga + 2, rows_a, sem_ga).start()

            for t in range(G):
                scat_start(jb, t, rows_b, sem_sb)
            return carry

        lax.fori_loop(0, ngrp, grp, 0)
        for t in range(G):
            scat_wait((ngrp - 1) * 2 * G + G, t, rows_b, sem_sb)
        plsc.subcore_barrier()
        pltpu.sync_copy(acc.at[pl.ds(s * RPT, RPT)],
                        out_hbm.at[c, pl.ds(s * RPT, RPT)])

    return scatter_kernel


def _row(f):
    return pl.BlockSpec((RB, f), lambda i: (i, 0))


def _full(r, c):
    return pl.BlockSpec((r, c), lambda i: (0, 0))


def _prep_body(dp, x, w1, dinv_o, z1_o):
    deg = jnp.sum(dp[...], axis=1, keepdims=True) + 1.0
    dinv = lax.rsqrt(deg)
    dinv_o[...] = dinv
    z1_o[...] = dinv * jnp.dot(x[...], w1[...],
                               preferred_element_type=jnp.float32)


_prep = pl.pallas_call(
    _prep_body,
    grid=(GRID,),
    in_specs=[_row(NW), _row(200), _full(200, 32)],
    out_specs=[_row(1), _row(32)],
    out_shape=[jax.ShapeDtypeStruct((N, 1), jnp.float32),
               jax.ShapeDtypeStruct((N, 32), jnp.float32)],
)


def _mid1_body(a0, a1, z1, dinv, b1, w2, z2_o):
    h = jnp.maximum(dinv[...] * (a0[...] + a1[...] + z1[...]) + b1[...], 0.0)
    z2_o[...] = dinv[...] * jnp.dot(h, w2[...],
                                    preferred_element_type=jnp.float32)


_mid1 = pl.pallas_call(
    _mid1_body,
    grid=(GRID,),
    in_specs=[_row(32), _row(32), _row(32), _row(1), _full(1, 32), _full(32, 16)],
    out_specs=_row(16),
    out_shape=jax.ShapeDtypeStruct((N, 16), jnp.float32),
)


def _mid2_body(a0, a1, z2, dinv, b2, z3_o):
    h = jnp.maximum(dinv[...] * (a0[...] + a1[...] + z2[...]) + b2[...], 0.0)
    z3_o[...] = dinv[...] * h


_mid2 = pl.pallas_call(
    _mid2_body,
    grid=(GRID,),
    in_specs=[_row(16), _row(16), _row(16), _row(1), _full(1, 16)],
    out_specs=_row(16),
    out_shape=jax.ShapeDtypeStruct((N, 16), jnp.float32),
)


def _final_body(a0, a1, z3, dinv, w3, b3, out_o):
    p = dinv[...] * (a0[...] + a1[...] + z3[...])
    h = jnp.dot(p, w3[...], preferred_element_type=jnp.float32) + b3[...]
    m = jnp.max(h, axis=1, keepdims=True)
    e = h - m
    out_o[...] = e - jnp.log(jnp.sum(jnp.exp(e), axis=1, keepdims=True))


_final = pl.pallas_call(
    _final_body,
    grid=(GRID,),
    in_specs=[_row(16), _row(16), _row(16), _row(1), _full(16, 200), _full(1, 200)],
    out_specs=_row(200),
    out_shape=jax.ShapeDtypeStruct((N, 200), jnp.float32),
)


def kernel(x, edge_index, W1, b1, W2, b2, W3, b3):
    e = edge_index.shape[1]
    nch = -(-(-(-e // (NW * CHUNK))) // (2 * G)) * (2 * G)
    ept = nch * CHUNK
    e_pad = ept * NW

    ei = edge_index.astype(jnp.int32)
    src = jnp.concatenate([ei[0], jnp.zeros((e_pad - e,), jnp.int32)])
    dst = jnp.concatenate([ei[1], jnp.full((e_pad - e,), N, jnp.int32)])
    src_r = src.reshape(NW, nch // G, WIDE)
    dst_r = dst.reshape(NW, nch, CHUNK)
    dst_f = dst.reshape(NW, ept)
    zdeg = jnp.zeros((DEGW,), jnp.float32)
    z16 = jnp.zeros((N_ACC, 16), jnp.float32)
    z32 = jnp.zeros((N_ACC, 32), jnp.float32)

    degp = _make_deg(ept)(dst_f, zdeg)
    dp = degp[:, :N].T

    pad32 = jnp.zeros((N_ACC - N, 32), jnp.float32)
    pad16 = jnp.zeros((N_ACC - N, 16), jnp.float32)

    dinv, zr1 = _prep(dp, x, W1)

    a = _make_scatter(32, nch)(jnp.concatenate([zr1, pad32]), src_r, dst_r, z32)
    zr2 = _mid1(a[0, :N], a[1, :N], zr1, dinv, b1.reshape(1, 32), W2)

    a = _make_scatter(16, nch)(jnp.concatenate([zr2, pad16]), src_r, dst_r, z16)
    zr3 = _mid2(a[0, :N], a[1, :N], zr2, dinv, b2.reshape(1, 16))

    a = _make_scatter(16, nch)(jnp.concatenate([zr3, pad16]), src_r, dst_r, z16)
    return _final(a[0, :N], a[1, :N], zr3, dinv, W3, b3.reshape(1, 200))


# speedup vs baseline: 48.2397x; 1.1054x over previous
import functools

import jax
import jax.numpy as jnp
from jax import lax
from jax.experimental import pallas as pl
from jax.experimental.pallas import tpu as pltpu
from jax.experimental.pallas import tpu_sc as plsc

N = 10000
NC, NS = 2, 16
NW = NC * NS
CHUNK = 128
RPT = 632
N_ACC = NS * RPT
DEG_ROWS = 640
DEG_RPT = DEG_ROWS // NS
DEG_CH = DEG_ROWS // CHUNK

RB = 1024
GRID = -(-N // RB)


def _sc_mesh():
    return plsc.VectorSubcoreMesh(core_axis_name="c", subcore_axis_name="s")


DEGW = DEG_ROWS * 16


def _make_deg(ept):
    steps = ept // 16

    @functools.partial(
        pl.kernel,
        out_type=jax.ShapeDtypeStruct((NW, DEGW), jnp.float32),
        mesh=_sc_mesh(),
        compiler_params=pltpu.CompilerParams(needs_layout_passes=False),
        scratch_types=[
            pltpu.VMEM((ept,), jnp.int32),
            pltpu.VMEM((DEGW,), jnp.float32),
        ],
    )
    def deg_kernel(dst_hbm, zeros_hbm, out_hbm, idxv, degv):
        c = lax.axis_index("c")
        s = lax.axis_index("s")
        wid = c * NS + s
        pltpu.sync_copy(zeros_hbm, degv)
        pltpu.sync_copy(dst_hbm.at[wid], idxv)
        ones = jnp.full((16,), 1.0, jnp.float32)

        def step(i, carry):
            idx = idxv[pl.ds(i * 16, 16)]
            plsc.addupdate_scatter(degv, [idx], ones)
            return carry

        lax.fori_loop(0, steps, step, 0)
        pltpu.sync_copy(degv, out_hbm.at[wid])

    return deg_kernel


G = 8
WIDE = G * CHUNK


def _make_scatter(f, nch):
    ngrp = nch // (2 * G)

    @functools.partial(
        pl.kernel,
        out_type=[jax.ShapeDtypeStruct((N_ACC, f), jnp.float32),
                  jax.ShapeDtypeStruct((N_ACC, f), jnp.float32)],
        mesh=_sc_mesh(),
        compiler_params=pltpu.CompilerParams(use_tc_tiling_on_sc=False),
        scratch_types=[
            pltpu.VMEM((nch // G, WIDE), jnp.int32),
            pltpu.VMEM((nch, CHUNK), jnp.int32),
            pltpu.VMEM((WIDE, f), jnp.float32),
            pltpu.VMEM((WIDE, f), jnp.float32),
            pltpu.VMEM_SHARED((N_ACC, f), jnp.float32),
            pltpu.VMEM_SHARED((N_ACC, f), jnp.float32),
            pltpu.SemaphoreType.DMA,
            pltpu.SemaphoreType.DMA,
            pltpu.SemaphoreType.DMA,
            pltpu.SemaphoreType.DMA,
        ],
    )
    def scatter_kernel(z_hbm, src_hbm, dst_hbm, zeros_hbm, out0_hbm, out1_hbm,
                       idxs, idxd, rows_a, rows_b, acc, zs,
                       sem_ga, sem_gb, sem_sa, sem_sb):
        c = lax.axis_index("c")
        s = lax.axis_index("s")
        wid = c * NS + s
        pltpu.sync_copy(zeros_hbm.at[pl.ds(s * RPT, RPT)],
                        acc.at[pl.ds(s * RPT, RPT)])
        pltpu.sync_copy(z_hbm.at[pl.ds(s * RPT, RPT)],
                        zs.at[pl.ds(s * RPT, RPT)])
        pltpu.sync_copy(src_hbm.at[wid], idxs)
        pltpu.sync_copy(dst_hbm.at[wid], idxd)
        plsc.subcore_barrier()

        def gath(g, buf, sem):
            return pltpu.make_async_copy(zs.at[idxs.at[g]], buf, sem)

        def scat_start(j, t, buf, sem):
            pltpu.async_copy(buf.at[pl.ds(t * CHUNK, CHUNK)],
                             acc.at[idxd.at[j + t]], sem, add=True)

        def scat_wait(j, t, buf, sem):
            pltpu.make_async_copy(buf.at[pl.ds(t * CHUNK, CHUNK)],
                                  acc.at[idxd.at[j + t]], sem).wait()

        gath(0, rows_a, sem_ga).start()

        def grp(k, carry):
            ga = 2 * k
            ja = ga * G
            jb = ja + G
            gath(ga, rows_a, sem_ga).wait()

            @pl.when(k > 0)
            def _():
                for t in range(G):
                    scat_wait(ja - G, t, rows_b, sem_sb)

            gath(ga + 1, rows_b, sem_gb).start()
            for t in range(G):
                scat_start(ja, t, rows_a, sem_sa)
            gath(ga + 1, rows_b, sem_gb).wait()
            for t in range(G):
                scat_wait(ja, t, rows_a, sem_sa)

            @pl.when(k + 1 < ngrp)
            def _():
                gath(ga + 2, rows_a, sem_ga).start()

            for t in range(G):
                scat_start(jb, t, rows_b, sem_sb)
            return carry

        lax.fori_loop(0, ngrp, grp, 0)
        for t in range(G):
            scat_wait((ngrp - 1) * 2 * G + G, t, rows_b, sem_sb)
        plsc.subcore_barrier()

        @pl.when(c == 0)
        def _():
            pltpu.sync_copy(acc.at[pl.ds(s * RPT, RPT)],
                            out0_hbm.at[pl.ds(s * RPT, RPT)])

        @pl.when(c == 1)
        def _():
            pltpu.sync_copy(acc.at[pl.ds(s * RPT, RPT)],
                            out1_hbm.at[pl.ds(s * RPT, RPT)])

    return scatter_kernel


def _row(f):
    return pl.BlockSpec((RB, f), lambda i: (i, 0))


def _full(r, c):
    return pl.BlockSpec((r, c), lambda i: (0, 0))


def _prep_body(dp, x, w1, dinv_o, z1_o):
    ones32 = jnp.full((NW, 1), 1.0, jnp.float32)
    deg = lax.dot_general(dp[...], ones32, (((0,), (0,)), ((), ())),
                          preferred_element_type=jnp.float32)
    dinv = lax.rsqrt(deg + 1.0)
    dinv_o[...] = dinv
    z1_o[...] = dinv * jnp.dot(x[...], w1[...],
                               preferred_element_type=jnp.float32)


_prep = pl.pallas_call(
    _prep_body,
    grid=(GRID,),
    in_specs=[pl.BlockSpec((NW, RB), lambda i: (0, i)), _row(200), _full(200, 32)],
    out_specs=[_row(1), _row(32)],
    out_shape=[jax.ShapeDtypeStruct((N, 1), jnp.float32),
               jax.ShapeDtypeStruct((N_ACC, 32), jnp.float32)],
)


def _mid1_body(a0, a1, z1, dinv, b1, w2, z2_o):
    h = jnp.maximum(dinv[...] * (a0[...] + a1[...] + z1[...]) + b1[...], 0.0)
    z2_o[...] = dinv[...] * jnp.dot(h, w2[...],
                                    preferred_element_type=jnp.float32)


_mid1 = pl.pallas_call(
    _mid1_body,
    grid=(GRID,),
    in_specs=[_row(32), _row(32), _row(32), _row(1), _full(1, 32), _full(32, 16)],
    out_specs=_row(16),
    out_shape=jax.ShapeDtypeStruct((N_ACC, 16), jnp.float32),
)


def _mid2_body(a0, a1, z2, dinv, b2, z3_o):
    h = jnp.maximum(dinv[...] * (a0[...] + a1[...] + z2[...]) + b2[...], 0.0)
    z3_o[...] = dinv[...] * h


_mid2 = pl.pallas_call(
    _mid2_body,
    grid=(GRID,),
    in_specs=[_row(16), _row(16), _row(16), _row(1), _full(1, 16)],
    out_specs=_row(16),
    out_shape=jax.ShapeDtypeStruct((N_ACC, 16), jnp.float32),
)


def _final_body(a0, a1, z3, dinv, w3, b3, out_o):
    p = dinv[...] * (a0[...] + a1[...] + z3[...])
    h = jnp.dot(p, w3[...], preferred_element_type=jnp.float32) + b3[...]
    m = jnp.max(h, axis=1, keepdims=True)
    e = h - m
    out_o[...] = e - jnp.log(jnp.sum(jnp.exp(e), axis=1, keepdims=True))


_final = pl.pallas_call(
    _final_body,
    grid=(GRID,),
    in_specs=[_row(16), _row(16), _row(16), _row(1), _full(16, 200), _full(1, 200)],
    out_specs=_row(200),
    out_shape=jax.ShapeDtypeStruct((N, 200), jnp.float32),
)


def kernel(x, edge_index, W1, b1, W2, b2, W3, b3):
    e = edge_index.shape[1]
    nch = -(-(-(-e // (NW * CHUNK))) // (2 * G)) * (2 * G)
    ept = nch * CHUNK
    e_pad = ept * NW

    ei = edge_index.astype(jnp.int32)
    src = jnp.concatenate([ei[0], jnp.zeros((e_pad - e,), jnp.int32)])
    dst = jnp.concatenate([ei[1], jnp.full((e_pad - e,), N, jnp.int32)])
    src_r = src.reshape(NW, nch // G, WIDE)
    dst_r = dst.reshape(NW, nch, CHUNK)
    dst_f = dst.reshape(NW, ept)
    zdeg = jnp.zeros((DEGW,), jnp.float32)
    z16 = jnp.zeros((N_ACC, 16), jnp.float32)
    z32 = jnp.zeros((N_ACC, 32), jnp.float32)

    degp = _make_deg(ept)(dst_f, zdeg)

    dinv, zr1 = _prep(degp, x, W1)

    a0, a1 = _make_scatter(32, nch)(zr1, src_r, dst_r, z32)
    zr2 = _mid1(a0, a1, zr1, dinv, b1.reshape(1, 32), W2)

    a0, a1 = _make_scatter(16, nch)(zr2, src_r, dst_r, z16)
    zr3 = _mid2(a0, a1, zr2, dinv, b2.reshape(1, 16))

    a0, a1 = _make_scatter(16, nch)(zr3, src_r, dst_r, z16)
    return _final(a0, a1, zr3, dinv, W3, b3.reshape(1, 200))
